# baseline (speedup 1.0000x reference)
import functools

import jax
import jax.numpy as jnp
from jax import lax
from jax.experimental import pallas as pl
from jax.experimental.pallas import tpu as pltpu
from jax.experimental.pallas import tpu_sc as plsc

N_NODES = 10000
IN_CH = 256
H1, C1 = 8, 8
D1 = H1 * C1
D2 = 128
W_ROW = 128

NC, NS, LANES = 2, 16, 16
NW = NC * NS
KCH = 64
NPAD = 10240
ROWS_PER_TILE = NPAD // NS
E_RAW = 160000
E_TOT = E_RAW + N_NODES
T_CHUNKS = -(-E_TOT // (NW * KCH))
EPAD = NW * KCH * T_CHUNKS
BLK = 256
N_BLOCKS = NPAD // BLK


def _dense1_body(x_ref, w_ref, asm_ref, adm_ref, st_ref, dt_ref):
    h = jnp.dot(x_ref[...], w_ref[...], preferred_element_type=jnp.float32)
    a_src = jnp.dot(h, asm_ref[...], preferred_element_type=jnp.float32)
    a_dst = jnp.dot(h, adm_ref[...], preferred_element_type=jnp.float32)
    st_ref[...] = jnp.concatenate([h, a_src], axis=1)
    dt_ref[...] = jnp.concatenate(
        [a_dst, jnp.zeros((BLK, W_ROW - D1), jnp.float32)], axis=1)


def _dense1(x_p, W1, AsrcExp, AdstExp):
    return pl.pallas_call(
        _dense1_body,
        grid=(N_BLOCKS,),
        in_specs=[
            pl.BlockSpec((BLK, IN_CH), lambda b: (b, 0)),
            pl.BlockSpec((IN_CH, D1), lambda b: (0, 0)),
            pl.BlockSpec((D1, D1), lambda b: (0, 0)),
            pl.BlockSpec((D1, D1), lambda b: (0, 0)),
        ],
        out_specs=[
            pl.BlockSpec((BLK, W_ROW), lambda b: (b, 0)),
            pl.BlockSpec((BLK, W_ROW), lambda b: (b, 0)),
        ],
        out_shape=[
            jax.ShapeDtypeStruct((NPAD, W_ROW), jnp.float32),
            jax.ShapeDtypeStruct((NPAD, W_ROW), jnp.float32),
        ],
    )(x_p, W1, AsrcExp, AdstExp)


def _dense2_body(acc_ref, b1_ref, w2_ref, as2_ref, ad2_ref,
                 h2_ref, asv_ref, adv_ref):
    blk = acc_ref[0] + acc_ref[1]
    num = blk[:, :D1]
    den = jnp.maximum(blk[:, D1:], 1e-30)
    g = jnp.maximum(num / den + b1_ref[...], 0.0)
    h2 = jnp.dot(g, w2_ref[...], preferred_element_type=jnp.float32)
    h2_ref[...] = h2
    asv_ref[...] = jnp.sum(h2 * as2_ref[...], axis=1)
    adv_ref[...] = jnp.sum(h2 * ad2_ref[...], axis=1)


def _dense2(acc1, b1, W2, att_src2, att_dst2):
    return pl.pallas_call(
        _dense2_body,
        grid=(N_BLOCKS,),
        in_specs=[
            pl.BlockSpec((NC, BLK, W_ROW), lambda b: (0, b, 0)),
            pl.BlockSpec((1, D1), lambda b: (0, 0)),
            pl.BlockSpec((D1, D2), lambda b: (0, 0)),
            pl.BlockSpec((1, D2), lambda b: (0, 0)),
            pl.BlockSpec((1, D2), lambda b: (0, 0)),
        ],
        out_specs=[
            pl.BlockSpec((BLK, D2), lambda b: (b, 0)),
            pl.BlockSpec((BLK,), lambda b: (b,)),
            pl.BlockSpec((BLK,), lambda b: (b,)),
        ],
        out_shape=[
            jax.ShapeDtypeStruct((NPAD, D2), jnp.float32),
            jax.ShapeDtypeStruct((NPAD,), jnp.float32),
            jax.ShapeDtypeStruct((NPAD,), jnp.float32),
        ],
    )(acc1, b1, W2, att_src2, att_dst2)


def _final_body(acc_ref, den_ref, b2_ref, out_ref):
    num = acc_ref[0] + acc_ref[1]
    den = jnp.maximum(jnp.sum(den_ref[...], axis=(0, 1)), 1e-30)
    o = num / den[:, None] + b2_ref[...]
    m = jnp.max(o, axis=1, keepdims=True)
    z = o - m
    out_ref[...] = z - jnp.log(jnp.sum(jnp.exp(z), axis=1, keepdims=True))


def _final(acc2, den2p, b2):
    return pl.pallas_call(
        _final_body,
        grid=(N_BLOCKS,),
        in_specs=[
            pl.BlockSpec((NC, BLK, D2), lambda b: (0, b, 0)),
            pl.BlockSpec((NC, NS, BLK), lambda b: (0, 0, b)),
            pl.BlockSpec((1, D2), lambda b: (0, 0)),
        ],
        out_specs=pl.BlockSpec((BLK, D2), lambda b: (b, 0)),
        out_shape=jax.ShapeDtypeStruct((NPAD, D2), jnp.float32),
    )(acc2, den2p, b2)


def _lrelu_exp(a):
    return jnp.exp(jnp.where(a >= 0.0, a, 0.2 * a))


@functools.partial(
    pl.kernel,
    out_type=jax.ShapeDtypeStruct((NC, NPAD, W_ROW), jnp.float32),
    mesh=plsc.VectorSubcoreMesh(core_axis_name="c", subcore_axis_name="s"),
    compiler_params=pltpu.CompilerParams(needs_layout_passes=False),
    scratch_types=[
        pltpu.VMEM((KCH,), jnp.int32),
        pltpu.VMEM((KCH,), jnp.int32),
        pltpu.VMEM((KCH,), jnp.int32),
        pltpu.VMEM((KCH,), jnp.int32),
        pltpu.VMEM((KCH, W_ROW), jnp.float32),
        pltpu.VMEM((KCH, W_ROW), jnp.float32),
        pltpu.VMEM((KCH, W_ROW), jnp.float32),
        pltpu.VMEM((KCH, W_ROW), jnp.float32),
        pltpu.SemaphoreType.DMA,
        pltpu.SemaphoreType.DMA,
        pltpu.SemaphoreType.DMA,
        pltpu.SemaphoreType.DMA,
        pltpu.VMEM_SHARED((NPAD, W_ROW), jnp.float32),
    ],
)
def _edge1(src_hbm, dst_hbm, st_hbm, dt_hbm, acc_out,
           srcv0, srcv1, dstv0, dstv1, gs0, gs1, gd0, gd1,
           sem0, sem1, ssem0, ssem1, acc_sh):
    cid = lax.axis_index("c")
    sid = lax.axis_index("s")
    wid = sid * NC + cid
    base0 = wid * (T_CHUNKS * KCH)
    zv = jnp.zeros((LANES,), jnp.float32)
    srcv = (srcv0, srcv1)
    dstv = (dstv0, dstv1)
    gs = (gs0, gs1)
    gd = (gd0, gd1)
    sem = (sem0, sem1)
    ssem = (ssem0, ssem1)

    def zero_row(r, _):
        for j in range(W_ROW // LANES):
            gs0[r, pl.ds(LANES * j, LANES)] = zv
        return 0

    lax.fori_loop(0, KCH, zero_row, 0)
    for cnk in range(ROWS_PER_TILE // KCH):
        r0 = sid * ROWS_PER_TILE + cnk * KCH
        pltpu.sync_copy(gs0, acc_sh.at[pl.ds(r0, KCH), :])
    plsc.subcore_barrier()

    def issue(i, b):
        base = base0 + i * KCH
        pltpu.sync_copy(src_hbm.at[pl.ds(base, KCH)], srcv[b])
        pltpu.sync_copy(dst_hbm.at[pl.ds(base, KCH)], dstv[b])
        pltpu.async_copy(st_hbm.at[srcv[b]], gs[b], sem[b])
        pltpu.async_copy(dt_hbm.at[dstv[b]], gd[b], sem[b])

    def wait_gathers(b):
        pltpu.make_async_copy(st_hbm.at[srcv[b]], gs[b], sem[b]).wait()
        pltpu.make_async_copy(dt_hbm.at[dstv[b]], gd[b], sem[b]).wait()

    def compute(b):
        gsb, gdb = gs[b], gd[b]

        @plsc.parallel_loop(0, KCH, 1, unroll=4)
        def edge(k):
            for j in range(D1 // LANES):
                a = (gsb[k, pl.ds(D1 + LANES * j, LANES)]
                     + gdb[k, pl.ds(LANES * j, LANES)])
                e = _lrelu_exp(a)
                gsb[k, pl.ds(D1 + LANES * j, LANES)] = e
                gsb[k, pl.ds(LANES * j, LANES)] = (
                    gsb[k, pl.ds(LANES * j, LANES)] * e)

    def start_scatter(b):
        pltpu.async_copy(gs[b], acc_sh.at[dstv[b]], ssem[b], add=True)

    def wait_scatter(b):
        pltpu.make_async_copy(gs[b], acc_sh.at[dstv[b]], ssem[b]).wait()

    issue(0, 0)

    def pair(q, _):
        i0 = 2 * q

        @pl.when(q > 0)
        def _():
            wait_scatter(1)

        issue(i0 + 1, 1)
        wait_gathers(0)
        compute(0)
        start_scatter(0)
        wait_gathers(1)
        compute(1)
        wait_scatter(0)

        @pl.when(i0 + 2 < T_CHUNKS)
        def _():
            issue(i0 + 2, 0)

        start_scatter(1)
        return 0

    lax.fori_loop(0, T_CHUNKS // 2, pair, 0)
    wait_scatter(1)
    plsc.subcore_barrier()

    for cnk in range(ROWS_PER_TILE // KCH):
        r0 = sid * ROWS_PER_TILE + cnk * KCH
        pltpu.sync_copy(acc_sh.at[pl.ds(r0, KCH), :], gs0)
        pltpu.sync_copy(gs0, acc_out.at[cid, pl.ds(r0, KCH), :])


@functools.partial(
    pl.kernel,
    out_type=[
        jax.ShapeDtypeStruct((NC, NPAD, D2), jnp.float32),
        jax.ShapeDtypeStruct((NC, NS, NPAD), jnp.float32),
    ],
    mesh=plsc.VectorSubcoreMesh(core_axis_name="c", subcore_axis_name="s"),
    compiler_params=pltpu.CompilerParams(needs_layout_passes=False),
    scratch_types=[
        pltpu.VMEM((KCH,), jnp.int32),
        pltpu.VMEM((KCH,), jnp.int32),
        pltpu.VMEM((KCH,), jnp.int32),
        pltpu.VMEM((KCH,), jnp.int32),
        pltpu.VMEM((KCH, D2), jnp.float32),
        pltpu.VMEM((KCH, D2), jnp.float32),
        pltpu.VMEM((KCH,), jnp.float32),
        pltpu.VMEM((NPAD,), jnp.float32),
        pltpu.VMEM((NPAD,), jnp.float32),
        pltpu.VMEM((NPAD,), jnp.float32),
        pltpu.SemaphoreType.DMA,
        pltpu.SemaphoreType.DMA,
        pltpu.SemaphoreType.DMA,
        pltpu.SemaphoreType.DMA,
        pltpu.VMEM_SHARED((NPAD, D2), jnp.float32),
    ],
)
def _edge2(src_hbm, dst_hbm, h2_hbm, as2_hbm, ad2_hbm, acc_out, den_out,
           srcv0, srcv1, dstv0, dstv1, gs0, gs1, ebuf, as2v, ad2v, denv,
           sem0, sem1, ssem0, ssem1, acc_sh):
    cid = lax.axis_index("c")
    sid = lax.axis_index("s")
    wid = sid * NC + cid
    base0 = wid * (T_CHUNKS * KCH)
    zv = jnp.zeros((LANES,), jnp.float32)
    srcv = (srcv0, srcv1)
    dstv = (dstv0, dstv1)
    gs = (gs0, gs1)
    sem = (sem0, sem1)
    ssem = (ssem0, ssem1)

    pltpu.sync_copy(as2_hbm, as2v)
    pltpu.sync_copy(ad2_hbm, ad2v)

    def zero_den(r, _):
        denv[pl.ds(r * LANES, LANES)] = zv
        return 0

    lax.fori_loop(0, NPAD // LANES, zero_den, 0)

    def zero_row(r, _):
        for j in range(D2 // LANES):
            gs0[r, pl.ds(LANES * j, LANES)] = zv
        return 0

    lax.fori_loop(0, KCH, zero_row, 0)
    for cnk in range(ROWS_PER_TILE // KCH):
        r0 = sid * ROWS_PER_TILE + cnk * KCH
        pltpu.sync_copy(gs0, acc_sh.at[pl.ds(r0, KCH), :])
    plsc.subcore_barrier()

    def issue(i, b):
        base = base0 + i * KCH
        pltpu.sync_copy(src_hbm.at[pl.ds(base, KCH)], srcv[b])
        pltpu.sync_copy(dst_hbm.at[pl.ds(base, KCH)], dstv[b])
        pltpu.async_copy(h2_hbm.at[srcv[b]], gs[b], sem[b])

    def compute(b):
        gsb, srcb, dstb = gs[b], srcv[b], dstv[b]

        def grp(t, _):
            sv = srcb[pl.ds(t * LANES, LANES)]
            dv = dstb[pl.ds(t * LANES, LANES)]
            e = _lrelu_exp(plsc.load_gather(as2v, [sv])
                           + plsc.load_gather(ad2v, [dv]))
            ebuf[pl.ds(t * LANES, LANES)] = e
            plsc.addupdate_scatter(denv, [dv], e)
            return 0

        lax.fori_loop(0, KCH // LANES, grp, 0)
        pltpu.make_async_copy(h2_hbm.at[srcb], gsb, sem[b]).wait()

        @plsc.parallel_loop(0, KCH, 1, unroll=2)
        def edge(k):
            ebc = plsc.load_gather(ebuf, [jnp.full((LANES,), k, jnp.int32)])
            for j in range(D2 // LANES):
                gsb[k, pl.ds(LANES * j, LANES)] = (
                    gsb[k, pl.ds(LANES * j, LANES)] * ebc)

    def start_scatter(b):
        pltpu.async_copy(gs[b], acc_sh.at[dstv[b]], ssem[b], add=True)

    def wait_scatter(b):
        pltpu.make_async_copy(gs[b], acc_sh.at[dstv[b]], ssem[b]).wait()

    issue(0, 0)

    def pair(q, _):
        i0 = 2 * q

        @pl.when(q > 0)
        def _():
            wait_scatter(1)

        issue(i0 + 1, 1)
        compute(0)
        start_scatter(0)
        compute(1)
        wait_scatter(0)

        @pl.when(i0 + 2 < T_CHUNKS)
        def _():
            issue(i0 + 2, 0)

        start_scatter(1)
        return 0

    lax.fori_loop(0, T_CHUNKS // 2, pair, 0)
    wait_scatter(1)
    pltpu.sync_copy(denv, den_out.at[cid, sid, :])
    plsc.subcore_barrier()

    for cnk in range(ROWS_PER_TILE // KCH):
        r0 = sid * ROWS_PER_TILE + cnk * KCH
        pltpu.sync_copy(acc_sh.at[pl.ds(r0, KCH), :], gs0)
        pltpu.sync_copy(gs0, acc_out.at[cid, pl.ds(r0, KCH), :])


def kernel(x, edge_index, W1, att_src1, att_dst1, b1, W2, att_src2, att_dst2, b2):
    N = N_NODES
    loop = jnp.arange(N, dtype=jnp.int32)
    src = jnp.concatenate([edge_index[0].astype(jnp.int32), loop])
    dst = jnp.concatenate([edge_index[1].astype(jnp.int32), loop])
    pad = jnp.full((EPAD - E_TOT,), N, jnp.int32)
    src = jnp.concatenate([src, pad])
    dst = jnp.concatenate([dst, pad])

    x_p = jnp.pad(x, ((0, NPAD - N), (0, 0)))

    ar = jnp.arange(D1)
    same_head = ((ar // C1)[:, None] == (ar // C1)[None, :]).astype(jnp.float32)
    AsrcExp = same_head * att_src1.reshape(-1)[:, None]
    AdstExp = same_head * att_dst1.reshape(-1)[:, None]

    st1, dt1 = _dense1(x_p, W1, AsrcExp, AdstExp)
    acc1 = _edge1(src, dst, st1, dt1)
    h2, as2, ad2 = _dense2(acc1, b1.reshape(1, D1), W2,
                           att_src2.reshape(1, D2), att_dst2.reshape(1, D2))
    acc2, den2p = _edge2(src, dst, h2, as2, ad2)
    out = _final(acc2, den2p, b2.reshape(1, D2))
    return out[:N]


# speedup vs baseline: 1.0916x; 1.0916x over previous
import functools

import jax
import jax.numpy as jnp
from jax import lax
from jax.experimental import pallas as pl
from jax.experimental.pallas import tpu as pltpu
from jax.experimental.pallas import tpu_sc as plsc

N_NODES = 10000
IN_CH = 256
H1, C1 = 8, 8
D1 = H1 * C1
D2 = 128
W_ROW = 128

NC, NS, LANES = 2, 16, 16
NW = NC * NS
KCH = 64
NPAD = 10240
ROWS_PER_TILE = NPAD // NS
E_RAW = 160000
E_TOT = E_RAW + N_NODES
T_CHUNKS = -(-E_TOT // (NW * KCH))
EPAD = NW * KCH * T_CHUNKS
BLK = 256
N_BLOCKS = NPAD // BLK


def _dense1_body(x_ref, w_ref, asm_ref, adm_ref, st_ref, dt_ref):
    h = jnp.dot(x_ref[...], w_ref[...], preferred_element_type=jnp.float32)
    a_src = jnp.dot(h, asm_ref[...], preferred_element_type=jnp.float32)
    a_dst = jnp.dot(h, adm_ref[...], preferred_element_type=jnp.float32)
    st_ref[...] = jnp.concatenate([h, a_src], axis=1)
    dt_ref[...] = jnp.concatenate(
        [a_dst, jnp.zeros((BLK, W_ROW - D1), jnp.float32)], axis=1)


def _dense1(x_p, W1, AsrcExp, AdstExp):
    return pl.pallas_call(
        _dense1_body,
        grid=(N_BLOCKS,),
        in_specs=[
            pl.BlockSpec((BLK, IN_CH), lambda b: (b, 0)),
            pl.BlockSpec((IN_CH, D1), lambda b: (0, 0)),
            pl.BlockSpec((D1, D1), lambda b: (0, 0)),
            pl.BlockSpec((D1, D1), lambda b: (0, 0)),
        ],
        out_specs=[
            pl.BlockSpec((BLK, W_ROW), lambda b: (b, 0)),
            pl.BlockSpec((BLK, W_ROW), lambda b: (b, 0)),
        ],
        out_shape=[
            jax.ShapeDtypeStruct((NPAD, W_ROW), jnp.float32),
            jax.ShapeDtypeStruct((NPAD, W_ROW), jnp.float32),
        ],
    )(x_p, W1, AsrcExp, AdstExp)


def _dense2_body(acc_ref, b1_ref, w2_ref, as2_ref, ad2_ref,
                 h2_ref, asv_ref, adv_ref):
    blk = acc_ref[0] + acc_ref[1]
    num = blk[:, :D1]
    den = jnp.maximum(blk[:, D1:], 1e-30)
    g = jnp.maximum(num / den + b1_ref[...], 0.0)
    h2 = jnp.dot(g, w2_ref[...], preferred_element_type=jnp.float32)
    h2_ref[...] = h2
    asv_ref[...] = jnp.sum(h2 * as2_ref[...], axis=1)
    adv_ref[...] = jnp.sum(h2 * ad2_ref[...], axis=1)


def _dense2(acc1, b1, W2, att_src2, att_dst2):
    return pl.pallas_call(
        _dense2_body,
        grid=(N_BLOCKS,),
        in_specs=[
            pl.BlockSpec((NC, BLK, W_ROW), lambda b: (0, b, 0)),
            pl.BlockSpec((1, D1), lambda b: (0, 0)),
            pl.BlockSpec((D1, D2), lambda b: (0, 0)),
            pl.BlockSpec((1, D2), lambda b: (0, 0)),
            pl.BlockSpec((1, D2), lambda b: (0, 0)),
        ],
        out_specs=[
            pl.BlockSpec((BLK, D2), lambda b: (b, 0)),
            pl.BlockSpec((BLK,), lambda b: (b,)),
            pl.BlockSpec((BLK,), lambda b: (b,)),
        ],
        out_shape=[
            jax.ShapeDtypeStruct((NPAD, D2), jnp.float32),
            jax.ShapeDtypeStruct((NPAD,), jnp.float32),
            jax.ShapeDtypeStruct((NPAD,), jnp.float32),
        ],
    )(acc1, b1, W2, att_src2, att_dst2)


def _final_body(acc_ref, den_ref, b2_ref, out_ref):
    num = acc_ref[0] + acc_ref[1]
    den = jnp.maximum(jnp.sum(den_ref[...], axis=(0, 1)), 1e-30)
    o = num / den[:, None] + b2_ref[...]
    m = jnp.max(o, axis=1, keepdims=True)
    z = o - m
    out_ref[...] = z - jnp.log(jnp.sum(jnp.exp(z), axis=1, keepdims=True))


def _final(acc2, den2p, b2):
    return pl.pallas_call(
        _final_body,
        grid=(N_BLOCKS,),
        in_specs=[
            pl.BlockSpec((NC, BLK, D2), lambda b: (0, b, 0)),
            pl.BlockSpec((NC, NS, BLK), lambda b: (0, 0, b)),
            pl.BlockSpec((1, D2), lambda b: (0, 0)),
        ],
        out_specs=pl.BlockSpec((BLK, D2), lambda b: (b, 0)),
        out_shape=jax.ShapeDtypeStruct((NPAD, D2), jnp.float32),
    )(acc2, den2p, b2)


def _lrelu_exp(a):
    return jnp.exp(jnp.where(a >= 0.0, a, 0.2 * a))


@functools.partial(
    pl.kernel,
    out_type=jax.ShapeDtypeStruct((NC, NPAD, W_ROW), jnp.float32),
    mesh=plsc.VectorSubcoreMesh(core_axis_name="c", subcore_axis_name="s"),
    compiler_params=pltpu.CompilerParams(needs_layout_passes=False),
    scratch_types=[
        pltpu.VMEM((KCH,), jnp.int32),
        pltpu.VMEM((KCH,), jnp.int32),
        pltpu.VMEM((KCH,), jnp.int32),
        pltpu.VMEM((KCH,), jnp.int32),
        pltpu.VMEM((KCH, W_ROW), jnp.float32),
        pltpu.VMEM((KCH, W_ROW), jnp.float32),
        pltpu.VMEM((KCH, W_ROW), jnp.float32),
        pltpu.VMEM((KCH, W_ROW), jnp.float32),
        pltpu.SemaphoreType.DMA,
        pltpu.SemaphoreType.DMA,
        pltpu.VMEM_SHARED((NPAD, W_ROW), jnp.float32),
    ],
)
def _edge1(src_hbm, dst_hbm, st_hbm, dt_hbm, acc_out,
           srcv0, srcv1, dstv0, dstv1, gs0, gs1, gd0, gd1,
           sem0, sem1, acc_sh):
    cid = lax.axis_index("c")
    sid = lax.axis_index("s")
    wid = sid * NC + cid
    base0 = wid * (T_CHUNKS * KCH)
    zv = jnp.zeros((LANES,), jnp.float32)
    srcv = (srcv0, srcv1)
    dstv = (dstv0, dstv1)
    gs = (gs0, gs1)
    gd = (gd0, gd1)
    sem = (sem0, sem1)

    def zero_row(r, _):
        for j in range(W_ROW // LANES):
            gs0[r, pl.ds(LANES * j, LANES)] = zv
        return 0

    lax.fori_loop(0, KCH, zero_row, 0)
    for cnk in range(ROWS_PER_TILE // KCH):
        r0 = sid * ROWS_PER_TILE + cnk * KCH
        pltpu.sync_copy(gs0, acc_sh.at[pl.ds(r0, KCH), :])
    plsc.subcore_barrier()

    def issue(i, b):
        base = base0 + i * KCH
        pltpu.sync_copy(src_hbm.at[pl.ds(base, KCH)], srcv[b])
        pltpu.sync_copy(dst_hbm.at[pl.ds(base, KCH)], dstv[b])
        pltpu.async_copy(st_hbm.at[srcv[b]], gs[b], sem[b])
        pltpu.async_copy(dt_hbm.at[dstv[b]], gd[b], sem[b])

    def wait_gathers(b):
        pltpu.make_async_copy(st_hbm.at[srcv[b]], gs[b], sem[b]).wait()
        pltpu.make_async_copy(dt_hbm.at[dstv[b]], gd[b], sem[b]).wait()

    def compute(b):
        gsb, gdb = gs[b], gd[b]

        @plsc.parallel_loop(0, KCH, 1, unroll=8)
        def edge(k):
            for j in range(D1 // LANES):
                a = (gsb[k, pl.ds(D1 + LANES * j, LANES)]
                     + gdb[k, pl.ds(LANES * j, LANES)])
                e = _lrelu_exp(a)
                gsb[k, pl.ds(D1 + LANES * j, LANES)] = e
                gsb[k, pl.ds(LANES * j, LANES)] = (
                    gsb[k, pl.ds(LANES * j, LANES)] * e)

    issue(0, 0)

    def pair(q, _):
        i0 = 2 * q
        issue(i0 + 1, 1)
        wait_gathers(0)
        compute(0)
        pltpu.sync_copy(gs0, acc_sh.at[dstv0], add=True)

        @pl.when(i0 + 2 < T_CHUNKS)
        def _():
            issue(i0 + 2, 0)

        wait_gathers(1)
        compute(1)
        pltpu.sync_copy(gs1, acc_sh.at[dstv1], add=True)
        return 0

    lax.fori_loop(0, T_CHUNKS // 2, pair, 0)
    plsc.subcore_barrier()

    for cnk in range(ROWS_PER_TILE // KCH):
        r0 = sid * ROWS_PER_TILE + cnk * KCH
        pltpu.sync_copy(acc_sh.at[pl.ds(r0, KCH), :], gs0)
        pltpu.sync_copy(gs0, acc_out.at[cid, pl.ds(r0, KCH), :])


@functools.partial(
    pl.kernel,
    out_type=[
        jax.ShapeDtypeStruct((NC, NPAD, D2), jnp.float32),
        jax.ShapeDtypeStruct((NC, NS, NPAD), jnp.float32),
    ],
    mesh=plsc.VectorSubcoreMesh(core_axis_name="c", subcore_axis_name="s"),
    compiler_params=pltpu.CompilerParams(needs_layout_passes=False),
    scratch_types=[
        pltpu.VMEM((KCH,), jnp.int32),
        pltpu.VMEM((KCH,), jnp.int32),
        pltpu.VMEM((KCH,), jnp.int32),
        pltpu.VMEM((KCH,), jnp.int32),
        pltpu.VMEM((KCH, D2), jnp.float32),
        pltpu.VMEM((KCH, D2), jnp.float32),
        pltpu.VMEM((KCH,), jnp.float32),
        pltpu.VMEM((NPAD,), jnp.float32),
        pltpu.VMEM((NPAD,), jnp.float32),
        pltpu.VMEM((NPAD,), jnp.float32),
        pltpu.SemaphoreType.DMA,
        pltpu.SemaphoreType.DMA,
        pltpu.VMEM_SHARED((NPAD, D2), jnp.float32),
    ],
)
def _edge2(src_hbm, dst_hbm, h2_hbm, as2_hbm, ad2_hbm, acc_out, den_out,
           srcv0, srcv1, dstv0, dstv1, gs0, gs1, ebuf, as2v, ad2v, denv,
           sem0, sem1, acc_sh):
    cid = lax.axis_index("c")
    sid = lax.axis_index("s")
    wid = sid * NC + cid
    base0 = wid * (T_CHUNKS * KCH)
    zv = jnp.zeros((LANES,), jnp.float32)
    srcv = (srcv0, srcv1)
    dstv = (dstv0, dstv1)
    gs = (gs0, gs1)
    sem = (sem0, sem1)

    pltpu.sync_copy(as2_hbm, as2v)
    pltpu.sync_copy(ad2_hbm, ad2v)

    def zero_den(r, _):
        denv[pl.ds(r * LANES, LANES)] = zv
        return 0

    lax.fori_loop(0, NPAD // LANES, zero_den, 0)

    def zero_row(r, _):
        for j in range(D2 // LANES):
            gs0[r, pl.ds(LANES * j, LANES)] = zv
        return 0

    lax.fori_loop(0, KCH, zero_row, 0)
    for cnk in range(ROWS_PER_TILE // KCH):
        r0 = sid * ROWS_PER_TILE + cnk * KCH
        pltpu.sync_copy(gs0, acc_sh.at[pl.ds(r0, KCH), :])
    plsc.subcore_barrier()

    def issue(i, b):
        base = base0 + i * KCH
        pltpu.sync_copy(src_hbm.at[pl.ds(base, KCH)], srcv[b])
        pltpu.sync_copy(dst_hbm.at[pl.ds(base, KCH)], dstv[b])
        pltpu.async_copy(h2_hbm.at[srcv[b]], gs[b], sem[b])

    def compute(b):
        gsb, srcb, dstb = gs[b], srcv[b], dstv[b]

        def grp(t, _):
            sv = srcb[pl.ds(t * LANES, LANES)]
            dv = dstb[pl.ds(t * LANES, LANES)]
            e = _lrelu_exp(plsc.load_gather(as2v, [sv])
                           + plsc.load_gather(ad2v, [dv]))
            ebuf[pl.ds(t * LANES, LANES)] = e
            plsc.addupdate_scatter(denv, [dv], e)
            return 0

        lax.fori_loop(0, KCH // LANES, grp, 0)
        pltpu.make_async_copy(h2_hbm.at[srcb], gsb, sem[b]).wait()

        @plsc.parallel_loop(0, KCH, 1, unroll=4)
        def edge(k):
            ebc = plsc.load_gather(ebuf, [jnp.full((LANES,), k, jnp.int32)])
            for j in range(D2 // LANES):
                gsb[k, pl.ds(LANES * j, LANES)] = (
                    gsb[k, pl.ds(LANES * j, LANES)] * ebc)

    issue(0, 0)

    def pair(q, _):
        i0 = 2 * q
        issue(i0 + 1, 1)
        compute(0)
        pltpu.sync_copy(gs0, acc_sh.at[dstv0], add=True)

        @pl.when(i0 + 2 < T_CHUNKS)
        def _():
            issue(i0 + 2, 0)

        compute(1)
        pltpu.sync_copy(gs1, acc_sh.at[dstv1], add=True)
        return 0

    lax.fori_loop(0, T_CHUNKS // 2, pair, 0)
    pltpu.sync_copy(denv, den_out.at[cid, sid, :])
    plsc.subcore_barrier()

    for cnk in range(ROWS_PER_TILE // KCH):
        r0 = sid * ROWS_PER_TILE + cnk * KCH
        pltpu.sync_copy(acc_sh.at[pl.ds(r0, KCH), :], gs0)
        pltpu.sync_copy(gs0, acc_out.at[cid, pl.ds(r0, KCH), :])


def kernel(x, edge_index, W1, att_src1, att_dst1, b1, W2, att_src2, att_dst2, b2):
    N = N_NODES
    loop = jnp.arange(N, dtype=jnp.int32)
    src = jnp.concatenate([edge_index[0].astype(jnp.int32), loop])
    dst = jnp.concatenate([edge_index[1].astype(jnp.int32), loop])
    pad = jnp.full((EPAD - E_TOT,), N, jnp.int32)
    src = jnp.concatenate([src, pad])
    dst = jnp.concatenate([dst, pad])

    x_p = jnp.pad(x, ((0, NPAD - N), (0, 0)))

    ar = jnp.arange(D1)
    same_head = ((ar // C1)[:, None] == (ar // C1)[None, :]).astype(jnp.float32)
    AsrcExp = same_head * att_src1.reshape(-1)[:, None]
    AdstExp = same_head * att_dst1.reshape(-1)[:, None]

    st1, dt1 = _dense1(x_p, W1, AsrcExp, AdstExp)
    acc1 = _edge1(src, dst, st1, dt1)
    h2, as2, ad2 = _dense2(acc1, b1.reshape(1, D1), W2,
                           att_src2.reshape(1, D2), att_dst2.reshape(1, D2))
    acc2, den2p = _edge2(src, dst, h2, as2, ad2)
    out = _final(acc2, den2p, b2.reshape(1, D2))
    return out[:N]
